# Initial kernel scaffold; baseline (speedup 1.0000x reference)
import jax
import jax.numpy as jnp
from jax.experimental import pallas as pl


def kernel(x, edge_index, W_self0, W_neigh0, b0, W_self1, W_neigh1, b1, W_self2, W_neigh2, b2):
    raise NotImplementedError("write your pallas kernel here")


# speedup vs baseline: 6.1755x; 6.1755x over previous
import functools

import jax
import jax.numpy as jnp
from jax import lax
from jax.experimental import pallas as pl
from jax.experimental.pallas import tpu as pltpu
from jax.experimental.pallas import tpu_sc as plsc

N = 10000
E = 320000
NC, NS = 2, 16
NW = NC * NS
CHUNK = 128
NCHUNKS = E // CHUNK
ROWS_PER_TILE = 624
TAIL_R0 = ROWS_PER_TILE * NS
TAIL_ROWS = N - TAIL_R0
BLK = 1000


def _seg_sum_call(feat, src, dst, zfeat, zdeg, with_deg):
  W = feat.shape[1]
  mesh = plsc.VectorSubcoreMesh(core_axis_name="c", subcore_axis_name="s")

  out_type = [jax.ShapeDtypeStruct((NC, N, W), jnp.float32)]
  scratch = [
      pltpu.VMEM((CHUNK,), jnp.int32),
      pltpu.VMEM((CHUNK,), jnp.int32),
      pltpu.VMEM((CHUNK, W), jnp.float32),
      pltpu.VMEM_SHARED((N, W), jnp.float32),
      pltpu.SemaphoreType.DMA,
  ]
  if with_deg:
    out_type.append(jax.ShapeDtypeStruct((N,), jnp.float32))
    out_type.append(jax.ShapeDtypeStruct((N,), jnp.float32))
    scratch += [
        pltpu.VMEM((CHUNK,), jnp.float32),
        pltpu.VMEM_SHARED((N,), jnp.float32),
    ]

  def body(feat_h, src_h, dst_h, zfeat_h, zdeg_h, out_h, *rest):
    if with_deg:
      deg0_h, deg1_h, src_v, dst_v, rows_v, acc, sem, ones_v, deg_acc = rest
    else:
      src_v, dst_v, rows_v, acc, sem = rest
    cid = lax.axis_index("c")
    sid = lax.axis_index("s")
    wid = sid * NC + cid
    r0 = pl.multiple_of(sid * ROWS_PER_TILE, 8)

    pltpu.sync_copy(zfeat_h.at[pl.ds(r0, ROWS_PER_TILE)],
                    acc.at[pl.ds(r0, ROWS_PER_TILE)])
    @pl.when(sid == NS - 1)
    def _():
      pltpu.sync_copy(zfeat_h.at[pl.ds(TAIL_R0, TAIL_ROWS)],
                      acc.at[pl.ds(TAIL_R0, TAIL_ROWS)])
    if with_deg:
      @pl.when(sid == 0)
      def _():
        pltpu.sync_copy(zdeg_h, deg_acc)
      for j in range(CHUNK // 16):
        ones_v[pl.ds(j * 16, 16)] = jnp.ones((16,), jnp.float32)
    plsc.subcore_barrier()

    nbase = NCHUNKS // NW
    nextra = NCHUNKS - nbase * NW
    nmine = jnp.where(wid < nextra, nbase + 1, nbase)

    def step(i, carry):
      chunk = wid + i * NW
      off = pl.multiple_of(chunk * CHUNK, CHUNK)
      pltpu.sync_copy(src_h.at[pl.ds(off, CHUNK)], src_v)
      pltpu.sync_copy(dst_h.at[pl.ds(off, CHUNK)], dst_v)
      pltpu.async_copy(feat_h.at[src_v], rows_v, sem).wait()
      pltpu.sync_copy(rows_v, acc.at[dst_v], add=True)
      if with_deg:
        pltpu.sync_copy(ones_v, deg_acc.at[dst_v], add=True)
      return carry

    lax.fori_loop(0, nmine, step, jnp.int32(0))
    plsc.subcore_barrier()

    pltpu.sync_copy(acc.at[pl.ds(r0, ROWS_PER_TILE)],
                    out_h.at[cid, pl.ds(r0, ROWS_PER_TILE)])
    @pl.when(sid == NS - 1)
    def _():
      pltpu.sync_copy(acc.at[pl.ds(TAIL_R0, TAIL_ROWS)],
                      out_h.at[cid, pl.ds(TAIL_R0, TAIL_ROWS)])
    if with_deg:
      @pl.when(jnp.logical_and(sid == 0, cid == 0))
      def _():
        pltpu.sync_copy(deg_acc, deg0_h)
      @pl.when(jnp.logical_and(sid == 0, cid == 1))
      def _():
        pltpu.sync_copy(deg_acc, deg1_h)

  k = pl.kernel(body, out_type=out_type, mesh=mesh, scratch_types=scratch,
                name=f"sage_seg_sum_w{W}" + ("_deg" if with_deg else ""))
  return k(feat, src, dst, zfeat, zdeg)


def _mm(x, w):
  n, d = x.shape
  h = w.shape[1]
  return pl.pallas_call(
      lambda x_ref, w_ref, o_ref: o_ref.__setitem__(
          ..., jnp.dot(x_ref[...], w_ref[...],
                       preferred_element_type=jnp.float32)),
      grid=(n // BLK,),
      in_specs=[
          pl.BlockSpec((BLK, d), lambda i: (i, 0)),
          pl.BlockSpec((d, h), lambda i: (0, 0)),
      ],
      out_specs=pl.BlockSpec((BLK, h), lambda i: (i, 0)),
      out_shape=jax.ShapeDtypeStruct((n, h), jnp.float32),
  )(x, w)


def _combine(h, w_self, b, p0, p1, d0, d1, w_next, relu):
  n, d = h.shape
  hh = w_self.shape[1]

  def body(h_ref, ws_ref, b_ref, p0_ref, p1_ref, d0_ref, d1_ref, *rest):
    if w_next is not None:
      wn_ref, o_ref, pre_ref = rest
    else:
      (o_ref,) = rest
    deg = jnp.maximum(d0_ref[...] + d1_ref[...], 1.0)
    out = (jnp.dot(h_ref[...], ws_ref[...], preferred_element_type=jnp.float32)
           + b_ref[...] + (p0_ref[...] + p1_ref[...]) / deg)
    if relu:
      out = jnp.maximum(out, 0.0)
    o_ref[...] = out
    if w_next is not None:
      pre_ref[...] = jnp.dot(out, wn_ref[...],
                             preferred_element_type=jnp.float32)

  in_specs = [
      pl.BlockSpec((BLK, d), lambda i: (i, 0)),
      pl.BlockSpec((d, hh), lambda i: (0, 0)),
      pl.BlockSpec((1, hh), lambda i: (0, 0)),
      pl.BlockSpec((BLK, hh), lambda i: (i, 0)),
      pl.BlockSpec((BLK, hh), lambda i: (i, 0)),
      pl.BlockSpec((BLK, 1), lambda i: (i, 0)),
      pl.BlockSpec((BLK, 1), lambda i: (i, 0)),
  ]
  args = [h, w_self, b, p0, p1, d0, d1]
  out_shape = [jax.ShapeDtypeStruct((n, hh), jnp.float32)]
  out_specs = [pl.BlockSpec((BLK, hh), lambda i: (i, 0))]
  if w_next is not None:
    hn = w_next.shape[1]
    in_specs.append(pl.BlockSpec((hh, hn), lambda i: (0, 0)))
    args.append(w_next)
    out_shape.append(jax.ShapeDtypeStruct((n, hn), jnp.float32))
    out_specs.append(pl.BlockSpec((BLK, hn), lambda i: (i, 0)))

  res = pl.pallas_call(
      body,
      grid=(n // BLK,),
      in_specs=in_specs,
      out_specs=out_specs,
      out_shape=out_shape,
  )(*args)
  return res if w_next is not None else res[0]


def _final(h, w_self, b, p0, p1, d0, d1, w_neigh):
  n, d = h.shape
  c = w_self.shape[1]

  def body(h_ref, ws_ref, b_ref, p0_ref, p1_ref, d0_ref, d1_ref, wn_ref,
           o_ref):
    deg = jnp.maximum(d0_ref[...] + d1_ref[...], 1.0)
    h_neigh = (p0_ref[...] + p1_ref[...]) / deg
    o_ref[...] = (
        jnp.dot(h_ref[...], ws_ref[...], preferred_element_type=jnp.float32)
        + b_ref[...]
        + jnp.dot(h_neigh, wn_ref[...], preferred_element_type=jnp.float32))

  return pl.pallas_call(
      body,
      grid=(n // BLK,),
      in_specs=[
          pl.BlockSpec((BLK, d), lambda i: (i, 0)),
          pl.BlockSpec((d, c), lambda i: (0, 0)),
          pl.BlockSpec((1, c), lambda i: (0, 0)),
          pl.BlockSpec((BLK, d), lambda i: (i, 0)),
          pl.BlockSpec((BLK, d), lambda i: (i, 0)),
          pl.BlockSpec((BLK, 1), lambda i: (i, 0)),
          pl.BlockSpec((BLK, 1), lambda i: (i, 0)),
          pl.BlockSpec((d, c), lambda i: (0, 0)),
      ],
      out_specs=pl.BlockSpec((BLK, c), lambda i: (i, 0)),
      out_shape=jax.ShapeDtypeStruct((n, c), jnp.float32),
  )(h, w_self, b, p0, p1, d0, d1, w_neigh)


def kernel(x, edge_index, W_self0, W_neigh0, b0, W_self1, W_neigh1, b1,
           W_self2, W_neigh2, b2):
  src = edge_index[0]
  dst = edge_index[1]
  zf128 = jnp.zeros((N, 128), jnp.float32)
  zdeg = jnp.zeros((N,), jnp.float32)

  pre0 = _mm(x, W_neigh0)
  P1, deg0, deg1 = _seg_sum_call(pre0, src, dst, zf128, zdeg, with_deg=True)
  d0 = deg0.reshape(N, 1)
  d1 = deg1.reshape(N, 1)

  h1, pre1 = _combine(x, W_self0, b0.reshape(1, -1), P1[0], P1[1], d0, d1,
                      W_neigh1, relu=True)
  (P2,) = _seg_sum_call(pre1, src, dst, zf128, zdeg, with_deg=False)
  h2 = _combine(h1, W_self1, b1.reshape(1, -1), P2[0], P2[1], d0, d1,
                None, relu=True)
  (P3,) = _seg_sum_call(h2, src, dst, zf128, zdeg, with_deg=False)
  out = _final(h2, W_self2, b2.reshape(1, -1), P3[0], P3[1], d0, d1, W_neigh2)
  return out


# speedup vs baseline: 11.0087x; 1.7826x over previous
import functools

import jax
import jax.numpy as jnp
from jax import lax
from jax.experimental import pallas as pl
from jax.experimental.pallas import tpu as pltpu
from jax.experimental.pallas import tpu_sc as plsc

N = 10000
E = 320000
NC, NS = 2, 16
NW = NC * NS
EPW = E // NW
CHUNK = 80
CPW = EPW // CHUNK
ROWS_PER_TILE = 624
TAIL_R0 = ROWS_PER_TILE * NS
TAIL_ROWS = N - TAIL_R0
BLK = 1000


def _seg_sum_call(feat, src, dst, zfeat, zdeg, with_deg):
  W = feat.shape[1]
  mesh = plsc.VectorSubcoreMesh(core_axis_name="c", subcore_axis_name="s")

  out_type = [jax.ShapeDtypeStruct((NC, N, W), jnp.float32)]
  scratch = [
      pltpu.VMEM((EPW,), jnp.int32),
      pltpu.VMEM((CPW, CHUNK), jnp.int32),
      pltpu.VMEM((CHUNK, W), jnp.float32),
      pltpu.VMEM((CHUNK, W), jnp.float32),
      pltpu.VMEM_SHARED((N, W), jnp.float32),
      pltpu.SemaphoreType.DMA,
      pltpu.SemaphoreType.DMA,
  ]
  if with_deg:
    out_type.append(jax.ShapeDtypeStruct((N,), jnp.float32))
    out_type.append(jax.ShapeDtypeStruct((N,), jnp.float32))
    scratch += [
        pltpu.VMEM((CHUNK,), jnp.float32),
        pltpu.VMEM_SHARED((N,), jnp.float32),
    ]

  def body(feat_h, src_h, dst_h, zfeat_h, zdeg_h, out_h, *rest):
    if with_deg:
      (deg0_h, deg1_h, src_all, dst_all, rows0, rows1, acc, sem0, sem1,
       ones_v, deg_acc) = rest
    else:
      src_all, dst_all, rows0, rows1, acc, sem0, sem1 = rest
    cid = lax.axis_index("c")
    sid = lax.axis_index("s")
    wid = sid * NC + cid
    r0 = pl.multiple_of(sid * ROWS_PER_TILE, 8)

    pltpu.sync_copy(src_h.at[wid], src_all)
    pltpu.sync_copy(dst_h.at[wid], dst_all)

    pltpu.sync_copy(zfeat_h.at[pl.ds(r0, ROWS_PER_TILE)],
                    acc.at[pl.ds(r0, ROWS_PER_TILE)])
    @pl.when(sid == NS - 1)
    def _():
      pltpu.sync_copy(zfeat_h.at[pl.ds(TAIL_R0, TAIL_ROWS)],
                      acc.at[pl.ds(TAIL_R0, TAIL_ROWS)])
    if with_deg:
      @pl.when(sid == 0)
      def _():
        pltpu.sync_copy(zdeg_h, deg_acc)
      for j in range(CHUNK // 16):
        ones_v[pl.ds(j * 16, 16)] = jnp.ones((16,), jnp.float32)
    plsc.subcore_barrier()

    bufs = (rows0, rows1)
    sems = (sem0, sem1)

    def idx(j):
      return src_all.at[pl.ds(pl.multiple_of(j * CHUNK, CHUNK), CHUNK)]

    def start(j, b):
      pltpu.async_copy(feat_h.at[idx(j)], bufs[b], sems[b])

    def finish(j, b):
      pltpu.make_async_copy(feat_h.at[idx(j)], bufs[b], sems[b]).wait()
      pltpu.sync_copy(bufs[b], acc.at[dst_all.at[j]], add=True)
      if with_deg:
        pltpu.sync_copy(ones_v, deg_acc.at[dst_all.at[j]], add=True)

    start(0, 0)

    def step(i, carry):
      j = i * 2
      start(j + 1, 1)
      finish(j, 0)
      start(j + 2, 0)
      finish(j + 1, 1)
      return carry

    lax.fori_loop(0, (CPW - 1) // 2, step, jnp.int32(0))
    finish(CPW - 1, 0)
    plsc.subcore_barrier()

    pltpu.sync_copy(acc.at[pl.ds(r0, ROWS_PER_TILE)],
                    out_h.at[cid, pl.ds(r0, ROWS_PER_TILE)])
    @pl.when(sid == NS - 1)
    def _():
      pltpu.sync_copy(acc.at[pl.ds(TAIL_R0, TAIL_ROWS)],
                      out_h.at[cid, pl.ds(TAIL_R0, TAIL_ROWS)])
    if with_deg:
      @pl.when(jnp.logical_and(sid == 0, cid == 0))
      def _():
        pltpu.sync_copy(deg_acc, deg0_h)
      @pl.when(jnp.logical_and(sid == 0, cid == 1))
      def _():
        pltpu.sync_copy(deg_acc, deg1_h)

  k = pl.kernel(body, out_type=out_type, mesh=mesh, scratch_types=scratch,
                name=f"sage_seg_sum_w{W}" + ("_deg" if with_deg else ""))
  return k(feat, src, dst, zfeat, zdeg)


def _mm(x, w):
  n, d = x.shape
  h = w.shape[1]
  return pl.pallas_call(
      lambda x_ref, w_ref, o_ref: o_ref.__setitem__(
          ..., jnp.dot(x_ref[...], w_ref[...],
                       preferred_element_type=jnp.float32)),
      grid=(n // BLK,),
      in_specs=[
          pl.BlockSpec((BLK, d), lambda i: (i, 0)),
          pl.BlockSpec((d, h), lambda i: (0, 0)),
      ],
      out_specs=pl.BlockSpec((BLK, h), lambda i: (i, 0)),
      out_shape=jax.ShapeDtypeStruct((n, h), jnp.float32),
  )(x, w)


def _combine(h, w_self, b, p0, p1, d0, d1, w_next, relu):
  n, d = h.shape
  hh = w_self.shape[1]

  def body(h_ref, ws_ref, b_ref, p0_ref, p1_ref, d0_ref, d1_ref, *rest):
    if w_next is not None:
      wn_ref, o_ref, pre_ref = rest
    else:
      (o_ref,) = rest
    deg = jnp.maximum(d0_ref[...] + d1_ref[...], 1.0)
    out = (jnp.dot(h_ref[...], ws_ref[...], preferred_element_type=jnp.float32)
           + b_ref[...] + (p0_ref[...] + p1_ref[...]) / deg)
    if relu:
      out = jnp.maximum(out, 0.0)
    o_ref[...] = out
    if w_next is not None:
      pre_ref[...] = jnp.dot(out, wn_ref[...],
                             preferred_element_type=jnp.float32)

  in_specs = [
      pl.BlockSpec((BLK, d), lambda i: (i, 0)),
      pl.BlockSpec((d, hh), lambda i: (0, 0)),
      pl.BlockSpec((1, hh), lambda i: (0, 0)),
      pl.BlockSpec((BLK, hh), lambda i: (i, 0)),
      pl.BlockSpec((BLK, hh), lambda i: (i, 0)),
      pl.BlockSpec((BLK, 1), lambda i: (i, 0)),
      pl.BlockSpec((BLK, 1), lambda i: (i, 0)),
  ]
  args = [h, w_self, b, p0, p1, d0, d1]
  out_shape = [jax.ShapeDtypeStruct((n, hh), jnp.float32)]
  out_specs = [pl.BlockSpec((BLK, hh), lambda i: (i, 0))]
  if w_next is not None:
    hn = w_next.shape[1]
    in_specs.append(pl.BlockSpec((hh, hn), lambda i: (0, 0)))
    args.append(w_next)
    out_shape.append(jax.ShapeDtypeStruct((n, hn), jnp.float32))
    out_specs.append(pl.BlockSpec((BLK, hn), lambda i: (i, 0)))

  res = pl.pallas_call(
      body,
      grid=(n // BLK,),
      in_specs=in_specs,
      out_specs=out_specs,
      out_shape=out_shape,
  )(*args)
  return res if w_next is not None else res[0]


def _final(h, w_self, b, p0, p1, d0, d1, w_neigh):
  n, d = h.shape
  c = w_self.shape[1]

  def body(h_ref, ws_ref, b_ref, p0_ref, p1_ref, d0_ref, d1_ref, wn_ref,
           o_ref):
    deg = jnp.maximum(d0_ref[...] + d1_ref[...], 1.0)
    h_neigh = (p0_ref[...] + p1_ref[...]) / deg
    o_ref[...] = (
        jnp.dot(h_ref[...], ws_ref[...], preferred_element_type=jnp.float32)
        + b_ref[...]
        + jnp.dot(h_neigh, wn_ref[...], preferred_element_type=jnp.float32))

  return pl.pallas_call(
      body,
      grid=(n // BLK,),
      in_specs=[
          pl.BlockSpec((BLK, d), lambda i: (i, 0)),
          pl.BlockSpec((d, c), lambda i: (0, 0)),
          pl.BlockSpec((1, c), lambda i: (0, 0)),
          pl.BlockSpec((BLK, d), lambda i: (i, 0)),
          pl.BlockSpec((BLK, d), lambda i: (i, 0)),
          pl.BlockSpec((BLK, 1), lambda i: (i, 0)),
          pl.BlockSpec((BLK, 1), lambda i: (i, 0)),
          pl.BlockSpec((d, c), lambda i: (0, 0)),
      ],
      out_specs=pl.BlockSpec((BLK, c), lambda i: (i, 0)),
      out_shape=jax.ShapeDtypeStruct((n, c), jnp.float32),
  )(h, w_self, b, p0, p1, d0, d1, w_neigh)


def kernel(x, edge_index, W_self0, W_neigh0, b0, W_self1, W_neigh1, b1,
           W_self2, W_neigh2, b2):
  src = edge_index[0].reshape(NW, EPW)
  dst = edge_index[1].reshape(NW, CPW, CHUNK)
  zf128 = jnp.zeros((N, 128), jnp.float32)
  zdeg = jnp.zeros((N,), jnp.float32)

  pre0 = _mm(x, W_neigh0)
  P1, deg0, deg1 = _seg_sum_call(pre0, src, dst, zf128, zdeg, with_deg=True)
  d0 = deg0.reshape(N, 1)
  d1 = deg1.reshape(N, 1)

  h1, pre1 = _combine(x, W_self0, b0.reshape(1, -1), P1[0], P1[1], d0, d1,
                      W_neigh1, relu=True)
  (P2,) = _seg_sum_call(pre1, src, dst, zf128, zdeg, with_deg=False)
  h2 = _combine(h1, W_self1, b1.reshape(1, -1), P2[0], P2[1], d0, d1,
                None, relu=True)
  (P3,) = _seg_sum_call(h2, src, dst, zf128, zdeg, with_deg=False)
  out = _final(h2, W_self2, b2.reshape(1, -1), P3[0], P3[1], d0, d1, W_neigh2)
  return out


# speedup vs baseline: 12.0486x; 1.0945x over previous
import functools

import jax
import jax.numpy as jnp
from jax import lax
from jax.experimental import pallas as pl
from jax.experimental.pallas import tpu as pltpu
from jax.experimental.pallas import tpu_sc as plsc

N = 10000
E = 320000
NC, NS = 2, 16
NW = NC * NS
EPW = E // NW
CHUNK = 80
CPW = EPW // CHUNK
CPAD = CPW + 1
ROWS_PER_TILE = 624
TAIL_R0 = ROWS_PER_TILE * NS
TAIL_ROWS = N - TAIL_R0
BLK = 1000


def _seg_sum_call(feat, src, dst, zfeat, zdeg, with_deg):
  W = feat.shape[1]
  mesh = plsc.VectorSubcoreMesh(core_axis_name="c", subcore_axis_name="s")

  out_type = [jax.ShapeDtypeStruct((NC, N, W), jnp.float32)]
  scratch = (
      [pltpu.VMEM((CHUNK, W), jnp.float32)] * 4
      + [pltpu.VMEM((CHUNK,), jnp.int32)] * 8
      + [pltpu.VMEM((CHUNK,), jnp.int32)] * 8
      + [pltpu.VMEM_SHARED((N, W), jnp.float32)]
      + [pltpu.SemaphoreType.DMA] * 16
  )
  if with_deg:
    out_type.append(jax.ShapeDtypeStruct((N,), jnp.float32))
    out_type.append(jax.ShapeDtypeStruct((N,), jnp.float32))
    scratch += [
        pltpu.VMEM((CHUNK,), jnp.float32),
        pltpu.VMEM_SHARED((N,), jnp.float32),
    ]

  def body(feat_h, src_h, dst_h, zfeat_h, zdeg_h, out_h, *rest):
    if with_deg:
      deg0_h, deg1_h, rest = rest[0], rest[1], rest[2:]
      ones_v, deg_acc = rest[-2:]
      rest = rest[:-2]
    bufs = rest[0:4]
    sslots = rest[4:12]
    dslots = rest[12:20]
    acc = rest[20]
    gsems = rest[21:25]
    ssems = rest[25:29]
    isems = rest[29:37]
    cid = lax.axis_index("c")
    sid = lax.axis_index("s")
    wid = sid * NC + cid
    r0 = pl.multiple_of(sid * ROWS_PER_TILE, 8)
    ebase = pl.multiple_of(wid * (CPAD * CHUNK), 8)

    pltpu.sync_copy(zfeat_h.at[pl.ds(r0, ROWS_PER_TILE)],
                    acc.at[pl.ds(r0, ROWS_PER_TILE)])
    @pl.when(sid == NS - 1)
    def _():
      pltpu.sync_copy(zfeat_h.at[pl.ds(TAIL_R0, TAIL_ROWS)],
                      acc.at[pl.ds(TAIL_R0, TAIL_ROWS)])
    if with_deg:
      @pl.when(sid == 0)
      def _():
        pltpu.sync_copy(zdeg_h, deg_acc)
      for j in range(CHUNK // 16):
        ones_v[pl.ds(j * 16, 16)] = jnp.ones((16,), jnp.float32)
    plsc.subcore_barrier()

    def eoff(j):
      return pl.multiple_of(ebase + j * CHUNK, 8)

    def start_i(j, k):
      pltpu.async_copy(src_h.at[pl.ds(eoff(j), CHUNK)], sslots[k], isems[k])
      pltpu.async_copy(dst_h.at[pl.ds(eoff(j), CHUNK)], dslots[k], isems[k])

    def wait_i(j, k):
      pltpu.make_async_copy(src_h.at[pl.ds(eoff(j), CHUNK)], sslots[k],
                            isems[k]).wait()
      pltpu.make_async_copy(dst_h.at[pl.ds(eoff(j), CHUNK)], dslots[k],
                            isems[k]).wait()

    def start_g(j, b, k):
      pltpu.async_copy(feat_h.at[sslots[k]], bufs[b], gsems[b])

    def wait_g(j, b, k):
      pltpu.make_async_copy(feat_h.at[sslots[k]], bufs[b], gsems[b]).wait()

    def start_s(j, b, k):
      pltpu.async_copy(bufs[b], acc.at[dslots[k]], ssems[b], add=True)
      if with_deg:
        pltpu.sync_copy(ones_v, deg_acc.at[dslots[k]], add=True)

    def wait_s(j, b, k):
      pltpu.make_async_copy(bufs[b], acc.at[dslots[k]], ssems[b]).wait()

    def quad(c, cm8, first=False):
      def sl(k):
        return (cm8 + k) % 8
      if not first:
        wait_s(c - 2, 2, sl(-2))
      start_i(c + 6, sl(6))
      wait_i(c + 2, sl(2))
      start_g(c + 2, 2, sl(2))
      if not first:
        wait_s(c - 1, 3, sl(-1))
      start_i(c + 7, sl(7))
      wait_i(c + 3, sl(3))
      start_g(c + 3, 3, sl(3))
      wait_g(c, 0, sl(0))
      start_s(c, 0, sl(0))
      wait_g(c + 1, 1, sl(1))
      start_s(c + 1, 1, sl(1))
      wait_s(c, 0, sl(0))
      start_i(c + 8, sl(0))
      wait_i(c + 4, sl(4))
      start_g(c + 4, 0, sl(4))
      wait_s(c + 1, 1, sl(1))
      start_i(c + 9, sl(1))
      wait_i(c + 5, sl(5))
      start_g(c + 5, 1, sl(5))
      wait_g(c + 2, 2, sl(2))
      start_s(c + 2, 2, sl(2))
      wait_g(c + 3, 3, sl(3))
      start_s(c + 3, 3, sl(3))

    for j in range(6):
      start_i(j, j)
    wait_i(0, 0)
    start_g(0, 0, 0)
    wait_i(1, 1)
    start_g(1, 1, 1)
    quad(0, 0, first=True)

    def step(i, carry):
      c = i * 8 + 4
      quad(c, 4)
      quad(c + 4, 0)
      return carry

    lax.fori_loop(0, 14, step, jnp.int32(0))
    quad(116, 4)

    wait_s(118, 2, (120 - 2) % 8)
    wait_i(122, 122 % 8)
    start_g(122, 2, 122 % 8)
    wait_s(119, 3, 119 % 8)
    wait_i(123, 123 % 8)
    start_g(123, 3, 123 % 8)
    wait_g(120, 0, 120 % 8)
    start_s(120, 0, 120 % 8)
    wait_g(121, 1, 121 % 8)
    start_s(121, 1, 121 % 8)
    wait_s(120, 0, 120 % 8)
    wait_i(124, 124 % 8)
    start_g(124, 0, 124 % 8)
    wait_g(122, 2, 122 % 8)
    start_s(122, 2, 122 % 8)
    wait_g(123, 3, 123 % 8)
    start_s(123, 3, 123 % 8)
    wait_g(124, 0, 124 % 8)
    start_s(124, 0, 124 % 8)
    wait_i(125, 125 % 8)
    wait_s(121, 1, 121 % 8)
    wait_s(122, 2, 122 % 8)
    wait_s(123, 3, 123 % 8)
    wait_s(124, 0, 124 % 8)
    plsc.subcore_barrier()

    pltpu.sync_copy(acc.at[pl.ds(r0, ROWS_PER_TILE)],
                    out_h.at[cid, pl.ds(r0, ROWS_PER_TILE)])
    @pl.when(sid == NS - 1)
    def _():
      pltpu.sync_copy(acc.at[pl.ds(TAIL_R0, TAIL_ROWS)],
                      out_h.at[cid, pl.ds(TAIL_R0, TAIL_ROWS)])
    if with_deg:
      @pl.when(jnp.logical_and(sid == 0, cid == 0))
      def _():
        pltpu.sync_copy(deg_acc, deg0_h)
      @pl.when(jnp.logical_and(sid == 0, cid == 1))
      def _():
        pltpu.sync_copy(deg_acc, deg1_h)

  k = pl.kernel(body, out_type=out_type, mesh=mesh, scratch_types=scratch,
                name=f"sage_seg_sum_w{W}" + ("_deg" if with_deg else ""))
  return k(feat, src, dst, zfeat, zdeg)


def _mm(x, w):
  n, d = x.shape
  h = w.shape[1]
  return pl.pallas_call(
      lambda x_ref, w_ref, o_ref: o_ref.__setitem__(
          ..., jnp.dot(x_ref[...], w_ref[...],
                       preferred_element_type=jnp.float32)),
      grid=(n // BLK,),
      in_specs=[
          pl.BlockSpec((BLK, d), lambda i: (i, 0)),
          pl.BlockSpec((d, h), lambda i: (0, 0)),
      ],
      out_specs=pl.BlockSpec((BLK, h), lambda i: (i, 0)),
      out_shape=jax.ShapeDtypeStruct((n, h), jnp.float32),
  )(x, w)


def _combine(h, w_self, b, p0, p1, d0, d1, w_next, relu):
  n, d = h.shape
  hh = w_self.shape[1]

  def body(h_ref, ws_ref, b_ref, p0_ref, p1_ref, d0_ref, d1_ref, *rest):
    if w_next is not None:
      wn_ref, o_ref, pre_ref = rest
    else:
      (o_ref,) = rest
    deg = jnp.maximum(d0_ref[...] + d1_ref[...], 1.0)
    out = (jnp.dot(h_ref[...], ws_ref[...], preferred_element_type=jnp.float32)
           + b_ref[...] + (p0_ref[...] + p1_ref[...]) / deg)
    if relu:
      out = jnp.maximum(out, 0.0)
    o_ref[...] = out
    if w_next is not None:
      pre_ref[...] = jnp.dot(out, wn_ref[...],
                             preferred_element_type=jnp.float32)

  in_specs = [
      pl.BlockSpec((BLK, d), lambda i: (i, 0)),
      pl.BlockSpec((d, hh), lambda i: (0, 0)),
      pl.BlockSpec((1, hh), lambda i: (0, 0)),
      pl.BlockSpec((BLK, hh), lambda i: (i, 0)),
      pl.BlockSpec((BLK, hh), lambda i: (i, 0)),
      pl.BlockSpec((BLK, 1), lambda i: (i, 0)),
      pl.BlockSpec((BLK, 1), lambda i: (i, 0)),
  ]
  args = [h, w_self, b, p0, p1, d0, d1]
  out_shape = [jax.ShapeDtypeStruct((n, hh), jnp.float32)]
  out_specs = [pl.BlockSpec((BLK, hh), lambda i: (i, 0))]
  if w_next is not None:
    hn = w_next.shape[1]
    in_specs.append(pl.BlockSpec((hh, hn), lambda i: (0, 0)))
    args.append(w_next)
    out_shape.append(jax.ShapeDtypeStruct((n, hn), jnp.float32))
    out_specs.append(pl.BlockSpec((BLK, hn), lambda i: (i, 0)))

  res = pl.pallas_call(
      body,
      grid=(n // BLK,),
      in_specs=in_specs,
      out_specs=out_specs,
      out_shape=out_shape,
  )(*args)
  return res if w_next is not None else res[0]


def _final(h, w_self, b, p0, p1, d0, d1, w_neigh):
  n, d = h.shape
  c = w_self.shape[1]

  def body(h_ref, ws_ref, b_ref, p0_ref, p1_ref, d0_ref, d1_ref, wn_ref,
           o_ref):
    deg = jnp.maximum(d0_ref[...] + d1_ref[...], 1.0)
    h_neigh = (p0_ref[...] + p1_ref[...]) / deg
    o_ref[...] = (
        jnp.dot(h_ref[...], ws_ref[...], preferred_element_type=jnp.float32)
        + b_ref[...]
        + jnp.dot(h_neigh, wn_ref[...], preferred_element_type=jnp.float32))

  return pl.pallas_call(
      body,
      grid=(n // BLK,),
      in_specs=[
          pl.BlockSpec((BLK, d), lambda i: (i, 0)),
          pl.BlockSpec((d, c), lambda i: (0, 0)),
          pl.BlockSpec((1, c), lambda i: (0, 0)),
          pl.BlockSpec((BLK, d), lambda i: (i, 0)),
          pl.BlockSpec((BLK, d), lambda i: (i, 0)),
          pl.BlockSpec((BLK, 1), lambda i: (i, 0)),
          pl.BlockSpec((BLK, 1), lambda i: (i, 0)),
          pl.BlockSpec((d, c), lambda i: (0, 0)),
      ],
      out_specs=pl.BlockSpec((BLK, c), lambda i: (i, 0)),
      out_shape=jax.ShapeDtypeStruct((n, c), jnp.float32),
  )(h, w_self, b, p0, p1, d0, d1, w_neigh)


def kernel(x, edge_index, W_self0, W_neigh0, b0, W_self1, W_neigh1, b1,
           W_self2, W_neigh2, b2):
  pad = ((0, 0), (0, 1), (0, 0))
  src = jnp.pad(edge_index[0].reshape(NW, CPW, CHUNK), pad).reshape(-1)
  dst = jnp.pad(edge_index[1].reshape(NW, CPW, CHUNK), pad).reshape(-1)
  zf128 = jnp.zeros((N, 128), jnp.float32)
  zdeg = jnp.zeros((N,), jnp.float32)

  pre0 = _mm(x, W_neigh0)
  P1, deg0, deg1 = _seg_sum_call(pre0, src, dst, zf128, zdeg, with_deg=True)
  d0 = deg0.reshape(N, 1)
  d1 = deg1.reshape(N, 1)

  h1, pre1 = _combine(x, W_self0, b0.reshape(1, -1), P1[0], P1[1], d0, d1,
                      W_neigh1, relu=True)
  (P2,) = _seg_sum_call(pre1, src, dst, zf128, zdeg, with_deg=False)
  h2 = _combine(h1, W_self1, b1.reshape(1, -1), P2[0], P2[1], d0, d1,
                None, relu=True)
  (P3,) = _seg_sum_call(h2, src, dst, zf128, zdeg, with_deg=False)
  out = _final(h2, W_self2, b2.reshape(1, -1), P3[0], P3[1], d0, d1, W_neigh2)
  return out


# speedup vs baseline: 12.2842x; 1.0196x over previous
import functools

import jax
import jax.numpy as jnp
from jax import lax
from jax.experimental import pallas as pl
from jax.experimental.pallas import tpu as pltpu
from jax.experimental.pallas import tpu_sc as plsc

N = 10000
E = 320000
NC, NS = 2, 16
NW = NC * NS
EPW = E // NW
CHUNK = 80
CPW = EPW // CHUNK
CPAD = CPW + 1
ROWS_PER_TILE = 624
TAIL_R0 = ROWS_PER_TILE * NS
TAIL_ROWS = N - TAIL_R0
BLK = 2000


def _seg_sum_call(feat, src, dst, zfeat, zdeg, with_deg):
  W = feat.shape[1]
  mesh = plsc.VectorSubcoreMesh(core_axis_name="c", subcore_axis_name="s")

  out_type = [jax.ShapeDtypeStruct((NC, N, W), jnp.float32)]
  scratch = (
      [pltpu.VMEM((CHUNK, W), jnp.float32)] * 4
      + [pltpu.VMEM((CHUNK,), jnp.int32)] * 8
      + [pltpu.VMEM((CHUNK,), jnp.int32)] * 8
      + [pltpu.VMEM_SHARED((N, W), jnp.float32)]
      + [pltpu.SemaphoreType.DMA] * 16
  )
  if with_deg:
    out_type.append(jax.ShapeDtypeStruct((N,), jnp.float32))
    out_type.append(jax.ShapeDtypeStruct((N,), jnp.float32))
    scratch += [
        pltpu.VMEM((CHUNK,), jnp.float32),
        pltpu.VMEM_SHARED((N,), jnp.float32),
    ]

  def body(feat_h, src_h, dst_h, zfeat_h, zdeg_h, out_h, *rest):
    if with_deg:
      deg0_h, deg1_h, rest = rest[0], rest[1], rest[2:]
      ones_v, deg_acc = rest[-2:]
      rest = rest[:-2]
    bufs = rest[0:4]
    sslots = rest[4:12]
    dslots = rest[12:20]
    acc = rest[20]
    gsems = rest[21:25]
    ssems = rest[25:29]
    isems = rest[29:37]
    cid = lax.axis_index("c")
    sid = lax.axis_index("s")
    wid = sid * NC + cid
    r0 = pl.multiple_of(sid * ROWS_PER_TILE, 8)
    ebase = pl.multiple_of(wid * (CPAD * CHUNK), 8)

    pltpu.sync_copy(zfeat_h.at[pl.ds(r0, ROWS_PER_TILE)],
                    acc.at[pl.ds(r0, ROWS_PER_TILE)])
    @pl.when(sid == NS - 1)
    def _():
      pltpu.sync_copy(zfeat_h.at[pl.ds(TAIL_R0, TAIL_ROWS)],
                      acc.at[pl.ds(TAIL_R0, TAIL_ROWS)])
    if with_deg:
      @pl.when(sid == 0)
      def _():
        pltpu.sync_copy(zdeg_h, deg_acc)
      for j in range(CHUNK // 16):
        ones_v[pl.ds(j * 16, 16)] = jnp.ones((16,), jnp.float32)
    plsc.subcore_barrier()

    def eoff(j):
      return pl.multiple_of(ebase + j * CHUNK, 8)

    def start_i(j, k):
      pltpu.async_copy(src_h.at[pl.ds(eoff(j), CHUNK)], sslots[k], isems[k])
      pltpu.async_copy(dst_h.at[pl.ds(eoff(j), CHUNK)], dslots[k], isems[k])

    def wait_i(j, k):
      pltpu.make_async_copy(src_h.at[pl.ds(eoff(j), CHUNK)], sslots[k],
                            isems[k]).wait()
      pltpu.make_async_copy(dst_h.at[pl.ds(eoff(j), CHUNK)], dslots[k],
                            isems[k]).wait()

    def start_g(j, b, k):
      pltpu.async_copy(feat_h.at[sslots[k]], bufs[b], gsems[b])

    def wait_g(j, b, k):
      pltpu.make_async_copy(feat_h.at[sslots[k]], bufs[b], gsems[b]).wait()

    def start_s(j, b, k):
      pltpu.async_copy(bufs[b], acc.at[dslots[k]], ssems[b], add=True)
      if with_deg:
        pltpu.sync_copy(ones_v, deg_acc.at[dslots[k]], add=True)

    def wait_s(j, b, k):
      pltpu.make_async_copy(bufs[b], acc.at[dslots[k]], ssems[b]).wait()

    def quad(c, cm8, first=False):
      def sl(k):
        return (cm8 + k) % 8
      if not first:
        wait_s(c - 2, 2, sl(-2))
      start_i(c + 6, sl(6))
      wait_i(c + 2, sl(2))
      start_g(c + 2, 2, sl(2))
      if not first:
        wait_s(c - 1, 3, sl(-1))
      start_i(c + 7, sl(7))
      wait_i(c + 3, sl(3))
      start_g(c + 3, 3, sl(3))
      wait_g(c, 0, sl(0))
      start_s(c, 0, sl(0))
      wait_g(c + 1, 1, sl(1))
      start_s(c + 1, 1, sl(1))
      wait_s(c, 0, sl(0))
      start_i(c + 8, sl(0))
      wait_i(c + 4, sl(4))
      start_g(c + 4, 0, sl(4))
      wait_s(c + 1, 1, sl(1))
      start_i(c + 9, sl(1))
      wait_i(c + 5, sl(5))
      start_g(c + 5, 1, sl(5))
      wait_g(c + 2, 2, sl(2))
      start_s(c + 2, 2, sl(2))
      wait_g(c + 3, 3, sl(3))
      start_s(c + 3, 3, sl(3))

    for j in range(6):
      start_i(j, j)
    wait_i(0, 0)
    start_g(0, 0, 0)
    wait_i(1, 1)
    start_g(1, 1, 1)
    quad(0, 0, first=True)

    def step(i, carry):
      c = i * 8 + 4
      quad(c, 4)
      quad(c + 4, 0)
      return carry

    lax.fori_loop(0, 14, step, jnp.int32(0))
    quad(116, 4)

    wait_s(118, 2, (120 - 2) % 8)
    wait_i(122, 122 % 8)
    start_g(122, 2, 122 % 8)
    wait_s(119, 3, 119 % 8)
    wait_i(123, 123 % 8)
    start_g(123, 3, 123 % 8)
    wait_g(120, 0, 120 % 8)
    start_s(120, 0, 120 % 8)
    wait_g(121, 1, 121 % 8)
    start_s(121, 1, 121 % 8)
    wait_s(120, 0, 120 % 8)
    wait_i(124, 124 % 8)
    start_g(124, 0, 124 % 8)
    wait_g(122, 2, 122 % 8)
    start_s(122, 2, 122 % 8)
    wait_g(123, 3, 123 % 8)
    start_s(123, 3, 123 % 8)
    wait_g(124, 0, 124 % 8)
    start_s(124, 0, 124 % 8)
    wait_i(125, 125 % 8)
    wait_s(121, 1, 121 % 8)
    wait_s(122, 2, 122 % 8)
    wait_s(123, 3, 123 % 8)
    wait_s(124, 0, 124 % 8)
    plsc.subcore_barrier()

    pltpu.sync_copy(acc.at[pl.ds(r0, ROWS_PER_TILE)],
                    out_h.at[cid, pl.ds(r0, ROWS_PER_TILE)])
    @pl.when(sid == NS - 1)
    def _():
      pltpu.sync_copy(acc.at[pl.ds(TAIL_R0, TAIL_ROWS)],
                      out_h.at[cid, pl.ds(TAIL_R0, TAIL_ROWS)])
    if with_deg:
      @pl.when(jnp.logical_and(sid == 0, cid == 0))
      def _():
        pltpu.sync_copy(deg_acc, deg0_h)
      @pl.when(jnp.logical_and(sid == 0, cid == 1))
      def _():
        pltpu.sync_copy(deg_acc, deg1_h)

  k = pl.kernel(body, out_type=out_type, mesh=mesh, scratch_types=scratch,
                name=f"sage_seg_sum_w{W}" + ("_deg" if with_deg else ""))
  return k(feat, src, dst, zfeat, zdeg)


def _mm(x, w):
  n, d = x.shape
  h = w.shape[1]
  return pl.pallas_call(
      lambda x_ref, w_ref, o_ref: o_ref.__setitem__(
          ..., jnp.dot(x_ref[...], w_ref[...],
                       preferred_element_type=jnp.float32)),
      grid=(n // BLK,),
      in_specs=[
          pl.BlockSpec((BLK, d), lambda i: (i, 0)),
          pl.BlockSpec((d, h), lambda i: (0, 0)),
      ],
      out_specs=pl.BlockSpec((BLK, h), lambda i: (i, 0)),
      out_shape=jax.ShapeDtypeStruct((n, h), jnp.float32),
  )(x, w)


def _combine(h, w_self, b, p0, p1, d0, d1, w_next, relu):
  n, d = h.shape
  hh = w_self.shape[1]

  def body(h_ref, ws_ref, b_ref, p0_ref, p1_ref, d0_ref, d1_ref, *rest):
    if w_next is not None:
      wn_ref, o_ref, pre_ref = rest
    else:
      (o_ref,) = rest
    deg = jnp.maximum(d0_ref[...] + d1_ref[...], 1.0)
    out = (jnp.dot(h_ref[...], ws_ref[...], preferred_element_type=jnp.float32)
           + b_ref[...] + (p0_ref[...] + p1_ref[...]) / deg)
    if relu:
      out = jnp.maximum(out, 0.0)
    o_ref[...] = out
    if w_next is not None:
      pre_ref[...] = jnp.dot(out, wn_ref[...],
                             preferred_element_type=jnp.float32)

  in_specs = [
      pl.BlockSpec((BLK, d), lambda i: (i, 0)),
      pl.BlockSpec((d, hh), lambda i: (0, 0)),
      pl.BlockSpec((1, hh), lambda i: (0, 0)),
      pl.BlockSpec((BLK, hh), lambda i: (i, 0)),
      pl.BlockSpec((BLK, hh), lambda i: (i, 0)),
      pl.BlockSpec((BLK, 1), lambda i: (i, 0)),
      pl.BlockSpec((BLK, 1), lambda i: (i, 0)),
  ]
  args = [h, w_self, b, p0, p1, d0, d1]
  out_shape = [jax.ShapeDtypeStruct((n, hh), jnp.float32)]
  out_specs = [pl.BlockSpec((BLK, hh), lambda i: (i, 0))]
  if w_next is not None:
    hn = w_next.shape[1]
    in_specs.append(pl.BlockSpec((hh, hn), lambda i: (0, 0)))
    args.append(w_next)
    out_shape.append(jax.ShapeDtypeStruct((n, hn), jnp.float32))
    out_specs.append(pl.BlockSpec((BLK, hn), lambda i: (i, 0)))

  res = pl.pallas_call(
      body,
      grid=(n // BLK,),
      in_specs=in_specs,
      out_specs=out_specs,
      out_shape=out_shape,
  )(*args)
  return res if w_next is not None else res[0]


def _final(h, w_self, b, p0, p1, d0, d1, w_neigh):
  n, d = h.shape
  c = w_self.shape[1]

  def body(h_ref, ws_ref, b_ref, p0_ref, p1_ref, d0_ref, d1_ref, wn_ref,
           o_ref):
    deg = jnp.maximum(d0_ref[...] + d1_ref[...], 1.0)
    h_neigh = (p0_ref[...] + p1_ref[...]) / deg
    o_ref[...] = (
        jnp.dot(h_ref[...], ws_ref[...], preferred_element_type=jnp.float32)
        + b_ref[...]
        + jnp.dot(h_neigh, wn_ref[...], preferred_element_type=jnp.float32))

  return pl.pallas_call(
      body,
      grid=(n // BLK,),
      in_specs=[
          pl.BlockSpec((BLK, d), lambda i: (i, 0)),
          pl.BlockSpec((d, c), lambda i: (0, 0)),
          pl.BlockSpec((1, c), lambda i: (0, 0)),
          pl.BlockSpec((BLK, d), lambda i: (i, 0)),
          pl.BlockSpec((BLK, d), lambda i: (i, 0)),
          pl.BlockSpec((BLK, 1), lambda i: (i, 0)),
          pl.BlockSpec((BLK, 1), lambda i: (i, 0)),
          pl.BlockSpec((d, c), lambda i: (0, 0)),
      ],
      out_specs=pl.BlockSpec((BLK, c), lambda i: (i, 0)),
      out_shape=jax.ShapeDtypeStruct((n, c), jnp.float32),
  )(h, w_self, b, p0, p1, d0, d1, w_neigh)


def kernel(x, edge_index, W_self0, W_neigh0, b0, W_self1, W_neigh1, b1,
           W_self2, W_neigh2, b2):
  pad = ((0, 0), (0, 1), (0, 0))
  src = jnp.pad(edge_index[0].reshape(NW, CPW, CHUNK), pad).reshape(-1)
  dst = jnp.pad(edge_index[1].reshape(NW, CPW, CHUNK), pad).reshape(-1)
  zf128 = jnp.zeros((N, 128), jnp.float32)
  zdeg = jnp.zeros((N,), jnp.float32)

  pre0 = _mm(x, W_neigh0)
  P1, deg0, deg1 = _seg_sum_call(pre0, src, dst, zf128, zdeg, with_deg=True)
  d0 = deg0.reshape(N, 1)
  d1 = deg1.reshape(N, 1)

  h1, pre1 = _combine(x, W_self0, b0.reshape(1, -1), P1[0], P1[1], d0, d1,
                      W_neigh1, relu=True)
  (P2,) = _seg_sum_call(pre1, src, dst, zf128, zdeg, with_deg=False)
  h2 = _combine(h1, W_self1, b1.reshape(1, -1), P2[0], P2[1], d0, d1,
                None, relu=True)
  (P3,) = _seg_sum_call(h2, src, dst, zf128, zdeg, with_deg=False)
  out = _final(h2, W_self2, b2.reshape(1, -1), P3[0], P3[1], d0, d1, W_neigh2)
  return out


# speedup vs baseline: 12.2997x; 1.0013x over previous
import functools

import jax
import jax.numpy as jnp
from jax import lax
from jax.experimental import pallas as pl
from jax.experimental.pallas import tpu as pltpu
from jax.experimental.pallas import tpu_sc as plsc

N = 10000
E = 320000
NC, NS = 2, 16
NW = NC * NS
EPW = E // NW
CHUNK = 80
CPW = EPW // CHUNK
CPAD = CPW + 1
ROWS_PER_TILE = 624
TAIL_R0 = ROWS_PER_TILE * NS
TAIL_ROWS = N - TAIL_R0
BLK = 10000


def _seg_sum_call(feat, src, dst, zfeat, zdeg, with_deg):
  W = feat.shape[1]
  mesh = plsc.VectorSubcoreMesh(core_axis_name="c", subcore_axis_name="s")

  out_type = [jax.ShapeDtypeStruct((NC, N, W), jnp.float32)]
  scratch = (
      [pltpu.VMEM((CHUNK, W), jnp.float32)] * 4
      + [pltpu.VMEM((CHUNK,), jnp.int32)] * 8
      + [pltpu.VMEM((CHUNK,), jnp.int32)] * 8
      + [pltpu.VMEM_SHARED((N, W), jnp.float32)]
      + [pltpu.SemaphoreType.DMA] * 16
  )
  if with_deg:
    out_type.append(jax.ShapeDtypeStruct((N,), jnp.float32))
    out_type.append(jax.ShapeDtypeStruct((N,), jnp.float32))
    scratch += [
        pltpu.VMEM((CHUNK,), jnp.float32),
        pltpu.VMEM_SHARED((N,), jnp.float32),
    ]

  def body(feat_h, src_h, dst_h, zfeat_h, zdeg_h, out_h, *rest):
    if with_deg:
      deg0_h, deg1_h, rest = rest[0], rest[1], rest[2:]
      ones_v, deg_acc = rest[-2:]
      rest = rest[:-2]
    bufs = rest[0:4]
    sslots = rest[4:12]
    dslots = rest[12:20]
    acc = rest[20]
    gsems = rest[21:25]
    ssems = rest[25:29]
    isems = rest[29:37]
    cid = lax.axis_index("c")
    sid = lax.axis_index("s")
    wid = sid * NC + cid
    r0 = pl.multiple_of(sid * ROWS_PER_TILE, 8)
    ebase = pl.multiple_of(wid * (CPAD * CHUNK), 8)

    pltpu.sync_copy(zfeat_h.at[pl.ds(r0, ROWS_PER_TILE)],
                    acc.at[pl.ds(r0, ROWS_PER_TILE)])
    @pl.when(sid == NS - 1)
    def _():
      pltpu.sync_copy(zfeat_h.at[pl.ds(TAIL_R0, TAIL_ROWS)],
                      acc.at[pl.ds(TAIL_R0, TAIL_ROWS)])
    if with_deg:
      @pl.when(sid == 0)
      def _():
        pltpu.sync_copy(zdeg_h, deg_acc)
      for j in range(CHUNK // 16):
        ones_v[pl.ds(j * 16, 16)] = jnp.ones((16,), jnp.float32)
    plsc.subcore_barrier()

    def eoff(j):
      return pl.multiple_of(ebase + j * CHUNK, 8)

    def start_i(j, k):
      pltpu.async_copy(src_h.at[pl.ds(eoff(j), CHUNK)], sslots[k], isems[k])
      pltpu.async_copy(dst_h.at[pl.ds(eoff(j), CHUNK)], dslots[k], isems[k])

    def wait_i(j, k):
      pltpu.make_async_copy(src_h.at[pl.ds(eoff(j), CHUNK)], sslots[k],
                            isems[k]).wait()
      pltpu.make_async_copy(dst_h.at[pl.ds(eoff(j), CHUNK)], dslots[k],
                            isems[k]).wait()

    def start_g(j, b, k):
      pltpu.async_copy(feat_h.at[sslots[k]], bufs[b], gsems[b])

    def wait_g(j, b, k):
      pltpu.make_async_copy(feat_h.at[sslots[k]], bufs[b], gsems[b]).wait()

    def start_s(j, b, k):
      pltpu.async_copy(bufs[b], acc.at[dslots[k]], ssems[b], add=True)
      if with_deg:
        pltpu.sync_copy(ones_v, deg_acc.at[dslots[k]], add=True)

    def wait_s(j, b, k):
      pltpu.make_async_copy(bufs[b], acc.at[dslots[k]], ssems[b]).wait()

    def quad(c, cm8, first=False):
      def sl(k):
        return (cm8 + k) % 8
      if not first:
        wait_s(c - 2, 2, sl(-2))
      start_i(c + 6, sl(6))
      wait_i(c + 2, sl(2))
      start_g(c + 2, 2, sl(2))
      if not first:
        wait_s(c - 1, 3, sl(-1))
      start_i(c + 7, sl(7))
      wait_i(c + 3, sl(3))
      start_g(c + 3, 3, sl(3))
      wait_g(c, 0, sl(0))
      start_s(c, 0, sl(0))
      wait_g(c + 1, 1, sl(1))
      start_s(c + 1, 1, sl(1))
      wait_s(c, 0, sl(0))
      start_i(c + 8, sl(0))
      wait_i(c + 4, sl(4))
      start_g(c + 4, 0, sl(4))
      wait_s(c + 1, 1, sl(1))
      start_i(c + 9, sl(1))
      wait_i(c + 5, sl(5))
      start_g(c + 5, 1, sl(5))
      wait_g(c + 2, 2, sl(2))
      start_s(c + 2, 2, sl(2))
      wait_g(c + 3, 3, sl(3))
      start_s(c + 3, 3, sl(3))

    for j in range(6):
      start_i(j, j)
    wait_i(0, 0)
    start_g(0, 0, 0)
    wait_i(1, 1)
    start_g(1, 1, 1)
    quad(0, 0, first=True)

    def step(i, carry):
      c = i * 8 + 4
      quad(c, 4)
      quad(c + 4, 0)
      return carry

    lax.fori_loop(0, 14, step, jnp.int32(0))
    quad(116, 4)

    wait_s(118, 2, (120 - 2) % 8)
    wait_i(122, 122 % 8)
    start_g(122, 2, 122 % 8)
    wait_s(119, 3, 119 % 8)
    wait_i(123, 123 % 8)
    start_g(123, 3, 123 % 8)
    wait_g(120, 0, 120 % 8)
    start_s(120, 0, 120 % 8)
    wait_g(121, 1, 121 % 8)
    start_s(121, 1, 121 % 8)
    wait_s(120, 0, 120 % 8)
    wait_i(124, 124 % 8)
    start_g(124, 0, 124 % 8)
    wait_g(122, 2, 122 % 8)
    start_s(122, 2, 122 % 8)
    wait_g(123, 3, 123 % 8)
    start_s(123, 3, 123 % 8)
    wait_g(124, 0, 124 % 8)
    start_s(124, 0, 124 % 8)
    wait_i(125, 125 % 8)
    wait_s(121, 1, 121 % 8)
    wait_s(122, 2, 122 % 8)
    wait_s(123, 3, 123 % 8)
    wait_s(124, 0, 124 % 8)
    plsc.subcore_barrier()

    pltpu.sync_copy(acc.at[pl.ds(r0, ROWS_PER_TILE)],
                    out_h.at[cid, pl.ds(r0, ROWS_PER_TILE)])
    @pl.when(sid == NS - 1)
    def _():
      pltpu.sync_copy(acc.at[pl.ds(TAIL_R0, TAIL_ROWS)],
                      out_h.at[cid, pl.ds(TAIL_R0, TAIL_ROWS)])
    if with_deg:
      @pl.when(jnp.logical_and(sid == 0, cid == 0))
      def _():
        pltpu.sync_copy(deg_acc, deg0_h)
      @pl.when(jnp.logical_and(sid == 0, cid == 1))
      def _():
        pltpu.sync_copy(deg_acc, deg1_h)

  k = pl.kernel(body, out_type=out_type, mesh=mesh, scratch_types=scratch,
                name=f"sage_seg_sum_w{W}" + ("_deg" if with_deg else ""))
  return k(feat, src, dst, zfeat, zdeg)


def _mm(x, w):
  n, d = x.shape
  h = w.shape[1]
  return pl.pallas_call(
      lambda x_ref, w_ref, o_ref: o_ref.__setitem__(
          ..., jnp.dot(x_ref[...], w_ref[...],
                       preferred_element_type=jnp.float32)),
      grid=(n // BLK,),
      in_specs=[
          pl.BlockSpec((BLK, d), lambda i: (i, 0)),
          pl.BlockSpec((d, h), lambda i: (0, 0)),
      ],
      out_specs=pl.BlockSpec((BLK, h), lambda i: (i, 0)),
      out_shape=jax.ShapeDtypeStruct((n, h), jnp.float32),
  )(x, w)


def _combine(h, w_self, b, p0, p1, d0, d1, w_next, relu):
  n, d = h.shape
  hh = w_self.shape[1]

  def body(h_ref, ws_ref, b_ref, p0_ref, p1_ref, d0_ref, d1_ref, *rest):
    if w_next is not None:
      wn_ref, o_ref, pre_ref = rest
    else:
      (o_ref,) = rest
    deg = jnp.maximum(d0_ref[...] + d1_ref[...], 1.0)
    out = (jnp.dot(h_ref[...], ws_ref[...], preferred_element_type=jnp.float32)
           + b_ref[...] + (p0_ref[...] + p1_ref[...]) / deg)
    if relu:
      out = jnp.maximum(out, 0.0)
    o_ref[...] = out
    if w_next is not None:
      pre_ref[...] = jnp.dot(out, wn_ref[...],
                             preferred_element_type=jnp.float32)

  in_specs = [
      pl.BlockSpec((BLK, d), lambda i: (i, 0)),
      pl.BlockSpec((d, hh), lambda i: (0, 0)),
      pl.BlockSpec((1, hh), lambda i: (0, 0)),
      pl.BlockSpec((BLK, hh), lambda i: (i, 0)),
      pl.BlockSpec((BLK, hh), lambda i: (i, 0)),
      pl.BlockSpec((BLK, 1), lambda i: (i, 0)),
      pl.BlockSpec((BLK, 1), lambda i: (i, 0)),
  ]
  args = [h, w_self, b, p0, p1, d0, d1]
  out_shape = [jax.ShapeDtypeStruct((n, hh), jnp.float32)]
  out_specs = [pl.BlockSpec((BLK, hh), lambda i: (i, 0))]
  if w_next is not None:
    hn = w_next.shape[1]
    in_specs.append(pl.BlockSpec((hh, hn), lambda i: (0, 0)))
    args.append(w_next)
    out_shape.append(jax.ShapeDtypeStruct((n, hn), jnp.float32))
    out_specs.append(pl.BlockSpec((BLK, hn), lambda i: (i, 0)))

  res = pl.pallas_call(
      body,
      grid=(n // BLK,),
      in_specs=in_specs,
      out_specs=out_specs,
      out_shape=out_shape,
  )(*args)
  return res if w_next is not None else res[0]


def _final(h, w_self, b, p0, p1, d0, d1, w_neigh):
  n, d = h.shape
  c = w_self.shape[1]

  def body(h_ref, ws_ref, b_ref, p0_ref, p1_ref, d0_ref, d1_ref, wn_ref,
           o_ref):
    deg = jnp.maximum(d0_ref[...] + d1_ref[...], 1.0)
    h_neigh = (p0_ref[...] + p1_ref[...]) / deg
    o_ref[...] = (
        jnp.dot(h_ref[...], ws_ref[...], preferred_element_type=jnp.float32)
        + b_ref[...]
        + jnp.dot(h_neigh, wn_ref[...], preferred_element_type=jnp.float32))

  return pl.pallas_call(
      body,
      grid=(n // BLK,),
      in_specs=[
          pl.BlockSpec((BLK, d), lambda i: (i, 0)),
          pl.BlockSpec((d, c), lambda i: (0, 0)),
          pl.BlockSpec((1, c), lambda i: (0, 0)),
          pl.BlockSpec((BLK, d), lambda i: (i, 0)),
          pl.BlockSpec((BLK, d), lambda i: (i, 0)),
          pl.BlockSpec((BLK, 1), lambda i: (i, 0)),
          pl.BlockSpec((BLK, 1), lambda i: (i, 0)),
          pl.BlockSpec((d, c), lambda i: (0, 0)),
      ],
      out_specs=pl.BlockSpec((BLK, c), lambda i: (i, 0)),
      out_shape=jax.ShapeDtypeStruct((n, c), jnp.float32),
  )(h, w_self, b, p0, p1, d0, d1, w_neigh)


def kernel(x, edge_index, W_self0, W_neigh0, b0, W_self1, W_neigh1, b1,
           W_self2, W_neigh2, b2):
  pad = ((0, 0), (0, 1), (0, 0))
  src = jnp.pad(edge_index[0].reshape(NW, CPW, CHUNK), pad).reshape(-1)
  dst = jnp.pad(edge_index[1].reshape(NW, CPW, CHUNK), pad).reshape(-1)
  zf128 = jnp.zeros((N, 128), jnp.float32)
  zdeg = jnp.zeros((N,), jnp.float32)

  pre0 = _mm(x, W_neigh0)
  P1, deg0, deg1 = _seg_sum_call(pre0, src, dst, zf128, zdeg, with_deg=True)
  d0 = deg0.reshape(N, 1)
  d1 = deg1.reshape(N, 1)

  h1, pre1 = _combine(x, W_self0, b0.reshape(1, -1), P1[0], P1[1], d0, d1,
                      W_neigh1, relu=True)
  (P2,) = _seg_sum_call(pre1, src, dst, zf128, zdeg, with_deg=False)
  h2 = _combine(h1, W_self1, b1.reshape(1, -1), P2[0], P2[1], d0, d1,
                None, relu=True)
  (P3,) = _seg_sum_call(h2, src, dst, zf128, zdeg, with_deg=False)
  out = _final(h2, W_self2, b2.reshape(1, -1), P3[0], P3[1], d0, d1, W_neigh2)
  return out


# speedup vs baseline: 12.4133x; 1.0092x over previous
import functools

import jax
import jax.numpy as jnp
from jax import lax
from jax.experimental import pallas as pl
from jax.experimental.pallas import tpu as pltpu
from jax.experimental.pallas import tpu_sc as plsc

N = 10000
E = 320000
NC, NS = 2, 16
NW = NC * NS
EPW = E // NW
CHUNK = 80
CPW = EPW // CHUNK
CPAD = CPW + 1
ROWS_PER_TILE = 624
TAIL_R0 = ROWS_PER_TILE * NS
TAIL_ROWS = N - TAIL_R0
BLK = 10000


def _seg_sum_call(feat, src, dst, zfeat, zdeg, with_deg):
  W = feat.shape[1]
  mesh = plsc.VectorSubcoreMesh(core_axis_name="c", subcore_axis_name="s")

  out_type = [jax.ShapeDtypeStruct((NC, N, W), jnp.float32)]
  scratch = (
      [pltpu.VMEM((CHUNK, W), jnp.float32)] * 4
      + [pltpu.VMEM((CHUNK,), jnp.int32)] * 8
      + [pltpu.VMEM((CHUNK,), jnp.int32)] * 8
      + [pltpu.VMEM_SHARED((N, W), jnp.float32)]
      + [pltpu.SemaphoreType.DMA] * 16
  )
  if with_deg:
    out_type.append(jax.ShapeDtypeStruct((N,), jnp.float32))
    out_type.append(jax.ShapeDtypeStruct((N,), jnp.float32))
    scratch += [
        pltpu.VMEM((CHUNK,), jnp.float32),
        pltpu.VMEM_SHARED((N,), jnp.float32),
    ]

  def body(feat_h, src_h, dst_h, zfeat_h, zdeg_h, out_h, *rest):
    if with_deg:
      deg0_h, deg1_h, rest = rest[0], rest[1], rest[2:]
      ones_v, deg_acc = rest[-2:]
      rest = rest[:-2]
    bufs = rest[0:4]
    sslots = rest[4:12]
    dslots = rest[12:20]
    acc = rest[20]
    gsems = rest[21:25]
    ssems = rest[25:29]
    isems = rest[29:37]
    cid = lax.axis_index("c")
    sid = lax.axis_index("s")
    wid = sid * NC + cid
    r0 = pl.multiple_of(sid * ROWS_PER_TILE, 8)
    ebase = pl.multiple_of(wid * (CPAD * CHUNK), 8)

    def eoff(j):
      return pl.multiple_of(ebase + j * CHUNK, 8)

    def start_i(j, k):
      pltpu.async_copy(src_h.at[pl.ds(eoff(j), CHUNK)], sslots[k], isems[k])
      pltpu.async_copy(dst_h.at[pl.ds(eoff(j), CHUNK)], dslots[k], isems[k])

    def wait_i(j, k):
      pltpu.make_async_copy(src_h.at[pl.ds(eoff(j), CHUNK)], sslots[k],
                            isems[k]).wait()
      pltpu.make_async_copy(dst_h.at[pl.ds(eoff(j), CHUNK)], dslots[k],
                            isems[k]).wait()

    def start_g(j, b, k):
      pltpu.async_copy(feat_h.at[sslots[k]], bufs[b], gsems[b])

    def wait_g(j, b, k):
      pltpu.make_async_copy(feat_h.at[sslots[k]], bufs[b], gsems[b]).wait()

    def start_s(j, b, k):
      pltpu.async_copy(bufs[b], acc.at[dslots[k]], ssems[b], add=True)
      if with_deg:
        pltpu.sync_copy(ones_v, deg_acc.at[dslots[k]], add=True)

    def wait_s(j, b, k):
      pltpu.make_async_copy(bufs[b], acc.at[dslots[k]], ssems[b]).wait()

    def quad(c, cm8, first=False):
      def sl(k):
        return (cm8 + k) % 8
      if not first:
        wait_s(c - 2, 2, sl(-2))
      start_i(c + 6, sl(6))
      wait_i(c + 2, sl(2))
      start_g(c + 2, 2, sl(2))
      if not first:
        wait_s(c - 1, 3, sl(-1))
      start_i(c + 7, sl(7))
      wait_i(c + 3, sl(3))
      start_g(c + 3, 3, sl(3))
      wait_g(c, 0, sl(0))
      start_s(c, 0, sl(0))
      wait_g(c + 1, 1, sl(1))
      start_s(c + 1, 1, sl(1))
      wait_s(c, 0, sl(0))
      start_i(c + 8, sl(0))
      wait_i(c + 4, sl(4))
      start_g(c + 4, 0, sl(4))
      wait_s(c + 1, 1, sl(1))
      start_i(c + 9, sl(1))
      wait_i(c + 5, sl(5))
      start_g(c + 5, 1, sl(5))
      wait_g(c + 2, 2, sl(2))
      start_s(c + 2, 2, sl(2))
      wait_g(c + 3, 3, sl(3))
      start_s(c + 3, 3, sl(3))

    for j in range(6):
      start_i(j, j)
    wait_i(0, 0)
    start_g(0, 0, 0)
    wait_i(1, 1)
    start_g(1, 1, 1)

    pltpu.sync_copy(zfeat_h.at[pl.ds(r0, ROWS_PER_TILE)],
                    acc.at[pl.ds(r0, ROWS_PER_TILE)])
    @pl.when(sid == NS - 1)
    def _():
      pltpu.sync_copy(zfeat_h.at[pl.ds(TAIL_R0, TAIL_ROWS)],
                      acc.at[pl.ds(TAIL_R0, TAIL_ROWS)])
    if with_deg:
      @pl.when(sid == 0)
      def _():
        pltpu.sync_copy(zdeg_h, deg_acc)
      for j in range(CHUNK // 16):
        ones_v[pl.ds(j * 16, 16)] = jnp.ones((16,), jnp.float32)
    plsc.subcore_barrier()

    quad(0, 0, first=True)

    def step(i, carry):
      c = i * 8 + 4
      quad(c, 4)
      quad(c + 4, 0)
      return carry

    lax.fori_loop(0, 14, step, jnp.int32(0))
    quad(116, 4)

    wait_s(118, 2, (120 - 2) % 8)
    wait_i(122, 122 % 8)
    start_g(122, 2, 122 % 8)
    wait_s(119, 3, 119 % 8)
    wait_i(123, 123 % 8)
    start_g(123, 3, 123 % 8)
    wait_g(120, 0, 120 % 8)
    start_s(120, 0, 120 % 8)
    wait_g(121, 1, 121 % 8)
    start_s(121, 1, 121 % 8)
    wait_s(120, 0, 120 % 8)
    wait_i(124, 124 % 8)
    start_g(124, 0, 124 % 8)
    wait_g(122, 2, 122 % 8)
    start_s(122, 2, 122 % 8)
    wait_g(123, 3, 123 % 8)
    start_s(123, 3, 123 % 8)
    wait_g(124, 0, 124 % 8)
    start_s(124, 0, 124 % 8)
    wait_i(125, 125 % 8)
    wait_s(121, 1, 121 % 8)
    wait_s(122, 2, 122 % 8)
    wait_s(123, 3, 123 % 8)
    wait_s(124, 0, 124 % 8)
    plsc.subcore_barrier()

    pltpu.sync_copy(acc.at[pl.ds(r0, ROWS_PER_TILE)],
                    out_h.at[cid, pl.ds(r0, ROWS_PER_TILE)])
    @pl.when(sid == NS - 1)
    def _():
      pltpu.sync_copy(acc.at[pl.ds(TAIL_R0, TAIL_ROWS)],
                      out_h.at[cid, pl.ds(TAIL_R0, TAIL_ROWS)])
    if with_deg:
      @pl.when(jnp.logical_and(sid == 0, cid == 0))
      def _():
        pltpu.sync_copy(deg_acc, deg0_h)
      @pl.when(jnp.logical_and(sid == 0, cid == 1))
      def _():
        pltpu.sync_copy(deg_acc, deg1_h)

  k = pl.kernel(body, out_type=out_type, mesh=mesh, scratch_types=scratch,
                name=f"sage_seg_sum_w{W}" + ("_deg" if with_deg else ""))
  return k(feat, src, dst, zfeat, zdeg)


def _mm(x, w):
  n, d = x.shape
  h = w.shape[1]
  return pl.pallas_call(
      lambda x_ref, w_ref, o_ref: o_ref.__setitem__(
          ..., jnp.dot(x_ref[...], w_ref[...],
                       preferred_element_type=jnp.float32)),
      grid=(n // BLK,),
      in_specs=[
          pl.BlockSpec((BLK, d), lambda i: (i, 0)),
          pl.BlockSpec((d, h), lambda i: (0, 0)),
      ],
      out_specs=pl.BlockSpec((BLK, h), lambda i: (i, 0)),
      out_shape=jax.ShapeDtypeStruct((n, h), jnp.float32),
  )(x, w)


def _combine(h, w_self, b, p0, p1, d0, d1, w_next, relu):
  n, d = h.shape
  hh = w_self.shape[1]

  def body(h_ref, ws_ref, b_ref, p0_ref, p1_ref, d0_ref, d1_ref, *rest):
    if w_next is not None:
      wn_ref, o_ref, pre_ref = rest
    else:
      (o_ref,) = rest
    deg = jnp.maximum(d0_ref[...] + d1_ref[...], 1.0)
    out = (jnp.dot(h_ref[...], ws_ref[...], preferred_element_type=jnp.float32)
           + b_ref[...] + (p0_ref[...] + p1_ref[...]) / deg)
    if relu:
      out = jnp.maximum(out, 0.0)
    o_ref[...] = out
    if w_next is not None:
      pre_ref[...] = jnp.dot(out, wn_ref[...],
                             preferred_element_type=jnp.float32)

  in_specs = [
      pl.BlockSpec((BLK, d), lambda i: (i, 0)),
      pl.BlockSpec((d, hh), lambda i: (0, 0)),
      pl.BlockSpec((1, hh), lambda i: (0, 0)),
      pl.BlockSpec((BLK, hh), lambda i: (i, 0)),
      pl.BlockSpec((BLK, hh), lambda i: (i, 0)),
      pl.BlockSpec((BLK, 1), lambda i: (i, 0)),
      pl.BlockSpec((BLK, 1), lambda i: (i, 0)),
  ]
  args = [h, w_self, b, p0, p1, d0, d1]
  out_shape = [jax.ShapeDtypeStruct((n, hh), jnp.float32)]
  out_specs = [pl.BlockSpec((BLK, hh), lambda i: (i, 0))]
  if w_next is not None:
    hn = w_next.shape[1]
    in_specs.append(pl.BlockSpec((hh, hn), lambda i: (0, 0)))
    args.append(w_next)
    out_shape.append(jax.ShapeDtypeStruct((n, hn), jnp.float32))
    out_specs.append(pl.BlockSpec((BLK, hn), lambda i: (i, 0)))

  res = pl.pallas_call(
      body,
      grid=(n // BLK,),
      in_specs=in_specs,
      out_specs=out_specs,
      out_shape=out_shape,
  )(*args)
  return res if w_next is not None else res[0]


def _final(h, w_self, b, p0, p1, d0, d1, w_neigh):
  n, d = h.shape
  c = w_self.shape[1]

  def body(h_ref, ws_ref, b_ref, p0_ref, p1_ref, d0_ref, d1_ref, wn_ref,
           o_ref):
    deg = jnp.maximum(d0_ref[...] + d1_ref[...], 1.0)
    h_neigh = (p0_ref[...] + p1_ref[...]) / deg
    o_ref[...] = (
        jnp.dot(h_ref[...], ws_ref[...], preferred_element_type=jnp.float32)
        + b_ref[...]
        + jnp.dot(h_neigh, wn_ref[...], preferred_element_type=jnp.float32))

  return pl.pallas_call(
      body,
      grid=(n // BLK,),
      in_specs=[
          pl.BlockSpec((BLK, d), lambda i: (i, 0)),
          pl.BlockSpec((d, c), lambda i: (0, 0)),
          pl.BlockSpec((1, c), lambda i: (0, 0)),
          pl.BlockSpec((BLK, d), lambda i: (i, 0)),
          pl.BlockSpec((BLK, d), lambda i: (i, 0)),
          pl.BlockSpec((BLK, 1), lambda i: (i, 0)),
          pl.BlockSpec((BLK, 1), lambda i: (i, 0)),
          pl.BlockSpec((d, c), lambda i: (0, 0)),
      ],
      out_specs=pl.BlockSpec((BLK, c), lambda i: (i, 0)),
      out_shape=jax.ShapeDtypeStruct((n, c), jnp.float32),
  )(h, w_self, b, p0, p1, d0, d1, w_neigh)


def kernel(x, edge_index, W_self0, W_neigh0, b0, W_self1, W_neigh1, b1,
           W_self2, W_neigh2, b2):
  pad = ((0, 0), (0, 1), (0, 0))
  src = jnp.pad(edge_index[0].reshape(NW, CPW, CHUNK), pad).reshape(-1)
  dst = jnp.pad(edge_index[1].reshape(NW, CPW, CHUNK), pad).reshape(-1)
  zf128 = jnp.zeros((N, 128), jnp.float32)
  zdeg = jnp.zeros((N,), jnp.float32)

  pre0 = _mm(x, W_neigh0)
  P1, deg0, deg1 = _seg_sum_call(pre0, src, dst, zf128, zdeg, with_deg=True)
  d0 = deg0.reshape(N, 1)
  d1 = deg1.reshape(N, 1)

  h1, pre1 = _combine(x, W_self0, b0.reshape(1, -1), P1[0], P1[1], d0, d1,
                      W_neigh1, relu=True)
  (P2,) = _seg_sum_call(pre1, src, dst, zf128, zdeg, with_deg=False)
  h2 = _combine(h1, W_self1, b1.reshape(1, -1), P2[0], P2[1], d0, d1,
                None, relu=True)
  (P3,) = _seg_sum_call(h2, src, dst, zf128, zdeg, with_deg=False)
  out = _final(h2, W_self2, b2.reshape(1, -1), P3[0], P3[1], d0, d1, W_neigh2)
  return out


# speedup vs baseline: 12.4388x; 1.0021x over previous
import jax
import jax.numpy as jnp
from jax import lax
from jax.experimental import pallas as pl
from jax.experimental.pallas import tpu as pltpu
from jax.experimental.pallas import tpu_sc as plsc

N = 10000
E = 320000
NC, NS = 2, 16
NW = NC * NS
EPW = E // NW
CHUNK = 80
CPW = EPW // CHUNK
CPAD = CPW + 1
ROWS_PER_TILE = 624
TAIL_R0 = ROWS_PER_TILE * NS
TAIL_ROWS = N - TAIL_R0
BLK = 10000


def _seg_sum_call(feat, src, dst, zfeat, zdeg, with_deg):
  W = feat.shape[1]
  mesh = plsc.VectorSubcoreMesh(core_axis_name="c", subcore_axis_name="s")

  out_type = [jax.ShapeDtypeStruct((NC, N, W), jnp.float32)]
  scratch = (
      [pltpu.VMEM((CHUNK, W), jnp.float32)] * 4
      + [pltpu.VMEM((CHUNK,), jnp.int32)] * 8
      + [pltpu.VMEM((CHUNK,), jnp.int32)] * 8
      + [pltpu.VMEM_SHARED((N, W), jnp.float32)]
      + [pltpu.SemaphoreType.DMA] * 16
  )
  if with_deg:
    out_type.append(jax.ShapeDtypeStruct((N,), jnp.float32))
    out_type.append(jax.ShapeDtypeStruct((N,), jnp.float32))
    scratch += [
        pltpu.VMEM((CHUNK,), jnp.float32),
        pltpu.VMEM_SHARED((N,), jnp.float32),
    ]

  def body(feat_h, src_h, dst_h, zfeat_h, zdeg_h, out_h, *rest):
    if with_deg:
      deg0_h, deg1_h, rest = rest[0], rest[1], rest[2:]
      ones_v, deg_acc = rest[-2:]
      rest = rest[:-2]
    bufs = rest[0:4]
    sslots = rest[4:12]
    dslots = rest[12:20]
    acc = rest[20]
    gsems = rest[21:25]
    ssems = rest[25:29]
    isems = rest[29:37]
    cid = lax.axis_index("c")
    sid = lax.axis_index("s")
    wid = sid * NC + cid
    r0 = pl.multiple_of(sid * ROWS_PER_TILE, 8)
    ebase = pl.multiple_of(wid * (CPAD * CHUNK), 8)

    def eoff(j):
      return pl.multiple_of(ebase + j * CHUNK, 8)

    def start_i(j, k):
      pltpu.async_copy(src_h.at[pl.ds(eoff(j), CHUNK)], sslots[k], isems[k])
      pltpu.async_copy(dst_h.at[pl.ds(eoff(j), CHUNK)], dslots[k], isems[k])

    def wait_i(j, k):
      pltpu.make_async_copy(src_h.at[pl.ds(eoff(j), CHUNK)], sslots[k],
                            isems[k]).wait()
      pltpu.make_async_copy(dst_h.at[pl.ds(eoff(j), CHUNK)], dslots[k],
                            isems[k]).wait()

    def start_g(j, b, k):
      pltpu.async_copy(feat_h.at[sslots[k]], bufs[b], gsems[b])

    def wait_g(j, b, k):
      pltpu.make_async_copy(feat_h.at[sslots[k]], bufs[b], gsems[b]).wait()

    def start_s(j, b, k):
      pltpu.async_copy(bufs[b], acc.at[dslots[k]], ssems[b], add=True)
      if with_deg:
        pltpu.sync_copy(ones_v, deg_acc.at[dslots[k]], add=True)

    def wait_s(j, b, k):
      pltpu.make_async_copy(bufs[b], acc.at[dslots[k]], ssems[b]).wait()

    def quad(c, cm8, first=False):
      def sl(k):
        return (cm8 + k) % 8
      if not first:
        wait_s(c - 2, 2, sl(-2))
      start_i(c + 6, sl(6))
      wait_i(c + 2, sl(2))
      start_g(c + 2, 2, sl(2))
      if not first:
        wait_s(c - 1, 3, sl(-1))
      start_i(c + 7, sl(7))
      wait_i(c + 3, sl(3))
      start_g(c + 3, 3, sl(3))
      wait_g(c, 0, sl(0))
      start_s(c, 0, sl(0))
      wait_g(c + 1, 1, sl(1))
      start_s(c + 1, 1, sl(1))
      wait_s(c, 0, sl(0))
      start_i(c + 8, sl(0))
      wait_i(c + 4, sl(4))
      start_g(c + 4, 0, sl(4))
      wait_s(c + 1, 1, sl(1))
      start_i(c + 9, sl(1))
      wait_i(c + 5, sl(5))
      start_g(c + 5, 1, sl(5))
      wait_g(c + 2, 2, sl(2))
      start_s(c + 2, 2, sl(2))
      wait_g(c + 3, 3, sl(3))
      start_s(c + 3, 3, sl(3))

    for j in range(6):
      start_i(j, j)
    wait_i(0, 0)
    start_g(0, 0, 0)
    wait_i(1, 1)
    start_g(1, 1, 1)

    pltpu.sync_copy(zfeat_h.at[pl.ds(r0, ROWS_PER_TILE)],
                    acc.at[pl.ds(r0, ROWS_PER_TILE)])
    @pl.when(sid == NS - 1)
    def _():
      pltpu.sync_copy(zfeat_h.at[pl.ds(TAIL_R0, TAIL_ROWS)],
                      acc.at[pl.ds(TAIL_R0, TAIL_ROWS)])
    if with_deg:
      @pl.when(sid == 0)
      def _():
        pltpu.sync_copy(zdeg_h, deg_acc)
      for j in range(CHUNK // 16):
        ones_v[pl.ds(j * 16, 16)] = jnp.ones((16,), jnp.float32)
    plsc.subcore_barrier()

    quad(0, 0, first=True)

    def step(i, carry):
      c = i * 8 + 4
      quad(c, 4)
      quad(c + 4, 0)
      return carry

    lax.fori_loop(0, 14, step, jnp.int32(0))
    quad(116, 4)

    wait_s(118, 2, (120 - 2) % 8)
    wait_i(122, 122 % 8)
    start_g(122, 2, 122 % 8)
    wait_s(119, 3, 119 % 8)
    wait_i(123, 123 % 8)
    start_g(123, 3, 123 % 8)
    wait_g(120, 0, 120 % 8)
    start_s(120, 0, 120 % 8)
    wait_g(121, 1, 121 % 8)
    start_s(121, 1, 121 % 8)
    wait_s(120, 0, 120 % 8)
    wait_i(124, 124 % 8)
    start_g(124, 0, 124 % 8)
    wait_g(122, 2, 122 % 8)
    start_s(122, 2, 122 % 8)
    wait_g(123, 3, 123 % 8)
    start_s(123, 3, 123 % 8)
    wait_g(124, 0, 124 % 8)
    start_s(124, 0, 124 % 8)
    wait_i(125, 125 % 8)
    wait_s(121, 1, 121 % 8)
    wait_s(122, 2, 122 % 8)
    wait_s(123, 3, 123 % 8)
    wait_s(124, 0, 124 % 8)
    plsc.subcore_barrier()

    pltpu.sync_copy(acc.at[pl.ds(r0, ROWS_PER_TILE)],
                    out_h.at[cid, pl.ds(r0, ROWS_PER_TILE)])
    @pl.when(sid == NS - 1)
    def _():
      pltpu.sync_copy(acc.at[pl.ds(TAIL_R0, TAIL_ROWS)],
                      out_h.at[cid, pl.ds(TAIL_R0, TAIL_ROWS)])
    if with_deg:
      @pl.when(jnp.logical_and(sid == 0, cid == 0))
      def _():
        pltpu.sync_copy(deg_acc, deg0_h)
      @pl.when(jnp.logical_and(sid == 0, cid == 1))
      def _():
        pltpu.sync_copy(deg_acc, deg1_h)

  k = pl.kernel(body, out_type=out_type, mesh=mesh, scratch_types=scratch,
                name=f"sage_seg_sum_w{W}" + ("_deg" if with_deg else ""))
  return k(feat, src, dst, zfeat, zdeg)


def _mm(x, w):
  n, d = x.shape
  h = w.shape[1]
  return pl.pallas_call(
      lambda x_ref, w_ref, o_ref: o_ref.__setitem__(
          ..., jnp.dot(x_ref[...], w_ref[...],
                       preferred_element_type=jnp.float32)),
      grid=(n // BLK,),
      in_specs=[
          pl.BlockSpec((BLK, d), lambda i: (i, 0)),
          pl.BlockSpec((d, h), lambda i: (0, 0)),
      ],
      out_specs=pl.BlockSpec((BLK, h), lambda i: (i, 0)),
      out_shape=jax.ShapeDtypeStruct((n, h), jnp.float32),
  )(x, w)


def _combine(h, w_self, b, p0, p1, d0, d1, w_next, relu):
  n, d = h.shape
  hh = w_self.shape[1]

  def body(h_ref, ws_ref, b_ref, p0_ref, p1_ref, d0_ref, d1_ref, *rest):
    if w_next is not None:
      wn_ref, o_ref, pre_ref = rest
    else:
      (o_ref,) = rest
    deg = jnp.maximum(d0_ref[...] + d1_ref[...], 1.0)
    out = (jnp.dot(h_ref[...], ws_ref[...], preferred_element_type=jnp.float32)
           + b_ref[...] + (p0_ref[...] + p1_ref[...]) / deg)
    if relu:
      out = jnp.maximum(out, 0.0)
    o_ref[...] = out
    if w_next is not None:
      pre_ref[...] = jnp.dot(out, wn_ref[...],
                             preferred_element_type=jnp.float32)

  in_specs = [
      pl.BlockSpec((BLK, d), lambda i: (i, 0)),
      pl.BlockSpec((d, hh), lambda i: (0, 0)),
      pl.BlockSpec((1, hh), lambda i: (0, 0)),
      pl.BlockSpec((BLK, hh), lambda i: (i, 0)),
      pl.BlockSpec((BLK, hh), lambda i: (i, 0)),
      pl.BlockSpec((BLK, 1), lambda i: (i, 0)),
      pl.BlockSpec((BLK, 1), lambda i: (i, 0)),
  ]
  args = [h, w_self, b, p0, p1, d0, d1]
  out_shape = [jax.ShapeDtypeStruct((n, hh), jnp.float32)]
  out_specs = [pl.BlockSpec((BLK, hh), lambda i: (i, 0))]
  if w_next is not None:
    hn = w_next.shape[1]
    in_specs.append(pl.BlockSpec((hh, hn), lambda i: (0, 0)))
    args.append(w_next)
    out_shape.append(jax.ShapeDtypeStruct((n, hn), jnp.float32))
    out_specs.append(pl.BlockSpec((BLK, hn), lambda i: (i, 0)))

  res = pl.pallas_call(
      body,
      grid=(n // BLK,),
      in_specs=in_specs,
      out_specs=out_specs,
      out_shape=out_shape,
  )(*args)
  return res if w_next is not None else res[0]


def _final(h, w_self, b, p0, p1, d0, d1, w_neigh):
  n, d = h.shape
  c = w_self.shape[1]

  def body(h_ref, ws_ref, b_ref, p0_ref, p1_ref, d0_ref, d1_ref, wn_ref,
           o_ref):
    deg = jnp.maximum(d0_ref[...] + d1_ref[...], 1.0)
    h_neigh = (p0_ref[...] + p1_ref[...]) / deg
    o_ref[...] = (
        jnp.dot(h_ref[...], ws_ref[...], preferred_element_type=jnp.float32)
        + b_ref[...]
        + jnp.dot(h_neigh, wn_ref[...], preferred_element_type=jnp.float32))

  return pl.pallas_call(
      body,
      grid=(n // BLK,),
      in_specs=[
          pl.BlockSpec((BLK, d), lambda i: (i, 0)),
          pl.BlockSpec((d, c), lambda i: (0, 0)),
          pl.BlockSpec((1, c), lambda i: (0, 0)),
          pl.BlockSpec((BLK, d), lambda i: (i, 0)),
          pl.BlockSpec((BLK, d), lambda i: (i, 0)),
          pl.BlockSpec((BLK, 1), lambda i: (i, 0)),
          pl.BlockSpec((BLK, 1), lambda i: (i, 0)),
          pl.BlockSpec((d, c), lambda i: (0, 0)),
      ],
      out_specs=pl.BlockSpec((BLK, c), lambda i: (i, 0)),
      out_shape=jax.ShapeDtypeStruct((n, c), jnp.float32),
  )(h, w_self, b, p0, p1, d0, d1, w_neigh)


def kernel(x, edge_index, W_self0, W_neigh0, b0, W_self1, W_neigh1, b1,
           W_self2, W_neigh2, b2):
  pad = ((0, 0), (0, 1), (0, 0))
  src = jnp.pad(edge_index[0].reshape(NW, CPW, CHUNK), pad).reshape(-1)
  dst = jnp.pad(edge_index[1].reshape(NW, CPW, CHUNK), pad).reshape(-1)
  zf128 = jnp.zeros((N, 128), jnp.float32)
  zdeg = jnp.zeros((N,), jnp.float32)

  pre0 = _mm(x, W_neigh0)
  P1, deg0, deg1 = _seg_sum_call(pre0, src, dst, zf128, zdeg, with_deg=True)
  d0 = deg0.reshape(N, 1)
  d1 = deg1.reshape(N, 1)

  h1, pre1 = _combine(x, W_self0, b0.reshape(1, -1), P1[0], P1[1], d0, d1,
                      W_neigh1, relu=True)
  (P2,) = _seg_sum_call(pre1, src, dst, zf128, zdeg, with_deg=False)
  h2 = _combine(h1, W_self1, b1.reshape(1, -1), P2[0], P2[1], d0, d1,
                None, relu=True)
  (P3,) = _seg_sum_call(h2, src, dst, zf128, zdeg, with_deg=False)
  out = _final(h2, W_self2, b2.reshape(1, -1), P3[0], P3[1], d0, d1, W_neigh2)
  return out
